# Initial kernel scaffold; baseline (speedup 1.0000x reference)
import jax
import jax.numpy as jnp
from jax.experimental import pallas as pl


def kernel(frames, Wfc, bfc):
    raise NotImplementedError("write your pallas kernel here")


# speedup vs baseline: 1.2158x; 1.2158x over previous
import functools

import jax
import jax.numpy as jnp
from jax import lax
from jax.experimental import pallas as pl
from jax.experimental.pallas import tpu as pltpu
from jax.experimental.pallas import tpu_sc as plsc

_B, _C, _T, _H, _W = 8, 3, 256, 64, 64
_PIX = _H * _W
_F = _B * _T
_NBINS = 512
_NW = 32
_FPW = _F // _NW
_LOOKUP = 101
_OUT_DIM = 128


def _sc_histograms(frames_flat):
    mesh = plsc.VectorSubcoreMesh(core_axis_name="c", subcore_axis_name="s")

    @functools.partial(
        pl.kernel,
        mesh=mesh,
        out_type=jax.ShapeDtypeStruct((_F * _NBINS,), jnp.float32),
        scratch_types=[
            pltpu.VMEM((_C * _PIX,), jnp.int32),
            pltpu.VMEM((_NBINS,), jnp.float32),
        ],
        compiler_params=pltpu.CompilerParams(needs_layout_passes=False),
    )
    def hist_kernel(frames_hbm, hist_hbm, plane_v, hist_v):
        wid = lax.axis_index("s") * 2 + lax.axis_index("c")
        f0 = wid * _FPW
        zeros16 = jnp.zeros((16,), jnp.float32)
        ones16 = jnp.ones((16,), jnp.float32)

        def frame_body(i, carry):
            f = f0 + i
            b = f // _T
            t = f - b * _T
            row = b * (_C * _T) + t
            for c in range(_C):
                pltpu.sync_copy(
                    frames_hbm.at[pl.ds((row + c * _T) * _PIX, _PIX)],
                    plane_v.at[pl.ds(c * _PIX, _PIX)])

            def zero_body(j, c):
                hist_v[pl.ds(j * 16, 16)] = zeros16
                return c

            lax.fori_loop(0, _NBINS // 16, zero_body, 0)

            def pix_body(g, c):
                for u in range(4):
                    off = (g * 4 + u) * 16
                    r = plane_v[pl.ds(off, 16)]
                    gg = plane_v[pl.ds(_PIX + off, 16)]
                    bb = plane_v[pl.ds(2 * _PIX + off, 16)]
                    bins = ((r & 0xE0) << 1) | ((gg & 0xE0) >> 2) | (bb >> 5)
                    plsc.addupdate_scatter(hist_v, [bins], ones16)
                return c

            lax.fori_loop(0, _PIX // 64, pix_body, 0)
            pltpu.sync_copy(hist_v, hist_hbm.at[pl.ds(f * _NBINS, _NBINS)])
            return carry

        lax.fori_loop(0, _FPW, frame_body, 0)

    return hist_kernel(frames_flat)


def _tc_post(hist3, w_pad, bias2):
    pad = (_LOOKUP - 1) // 2
    width = 512

    def body(x_ref, w_ref, b_ref, o_ref, sp_ref):
        x = x_ref[0]
        inv = 1.0 / jnp.sqrt(jnp.sum(x * x, axis=1, keepdims=True))
        xn = x * inv
        sims = lax.dot_general(xn, xn, (((1,), (1,)), ((), ())),
                               preferred_element_type=jnp.float32)
        sp_ref[...] = jnp.zeros((_T, width), jnp.float32)
        sp_ref[:, pad:pad + _T] = sims
        v = sp_ref[...]
        rows = lax.broadcasted_iota(jnp.int32, (_T, width), 0)
        for k in range(8):
            amt = 1 << k
            rolled = jnp.concatenate([v[:, amt:], v[:, :amt]], axis=1)
            v = jnp.where((rows & amt) != 0, rolled, v)
        band = v[:, :_OUT_DIM]
        out = lax.dot_general(band, w_ref[...], (((1,), (0,)), ((), ())),
                              preferred_element_type=jnp.float32)
        o_ref[0] = jnp.maximum(out + b_ref[...], 0.0)

    return pl.pallas_call(
        body,
        grid=(_B,),
        in_specs=[
            pl.BlockSpec((1, _T, _NBINS), lambda i: (i, 0, 0)),
            pl.BlockSpec((_OUT_DIM, _OUT_DIM), lambda i: (0, 0)),
            pl.BlockSpec((1, _OUT_DIM), lambda i: (0, 0)),
        ],
        out_specs=pl.BlockSpec((1, _T, _OUT_DIM), lambda i: (i, 0, 0)),
        out_shape=jax.ShapeDtypeStruct((_B, _T, _OUT_DIM), jnp.float32),
        scratch_shapes=[pltpu.VMEM((_T, width), jnp.float32)],
    )(hist3, w_pad, bias2)


def kernel(frames, Wfc, bfc):
    frames_flat = frames.reshape(_B * _C * _T * _PIX)
    hist = _sc_histograms(frames_flat)
    hist3 = hist.reshape(_B, _T, _NBINS)
    w_pad = jnp.pad(Wfc.T, ((0, _OUT_DIM - _LOOKUP), (0, 0)))
    bias2 = bfc.reshape(1, _OUT_DIM)
    return _tc_post(hist3, w_pad, bias2)


# speedup vs baseline: 2.1142x; 1.7389x over previous
import functools

import jax
import jax.numpy as jnp
from jax import lax
from jax.experimental import pallas as pl
from jax.experimental.pallas import tpu as pltpu
from jax.experimental.pallas import tpu_sc as plsc

_B, _C, _T, _H, _W = 8, 3, 256, 64, 64
_PIX = _H * _W
_F = _B * _T
_NBINS = 512
_NW = 32
_FPW = _F // _NW
_LOOKUP = 101
_OUT_DIM = 128


_PLANE3 = _C * _PIX


def _sc_histograms(frames_flat):
    mesh = plsc.VectorSubcoreMesh(core_axis_name="c", subcore_axis_name="s")

    @functools.partial(
        pl.kernel,
        mesh=mesh,
        out_type=jax.ShapeDtypeStruct((_F * _NBINS,), jnp.float32),
        scratch_types=[
            pltpu.VMEM((2 * _PLANE3,), jnp.int32),
            pltpu.VMEM((2 * _NBINS,), jnp.float32),
            pltpu.SemaphoreType.DMA,
            pltpu.SemaphoreType.DMA,
            pltpu.SemaphoreType.DMA,
            pltpu.SemaphoreType.DMA,
        ],
        compiler_params=pltpu.CompilerParams(needs_layout_passes=False),
    )
    def hist_kernel(frames_hbm, hist_hbm, plane_v, hist_v,
                    in_sem0, in_sem1, out_sem0, out_sem1):
        wid = lax.axis_index("s") * 2 + lax.axis_index("c")
        f0 = wid * _FPW
        zeros16 = jnp.zeros((16,), jnp.float32)
        ones16 = jnp.ones((16,), jnp.float32)

        def in_copies(f, base, sem):
            b = f // _T
            t = f - b * _T
            row = b * (_C * _T) + t
            return [
                pltpu.make_async_copy(
                    frames_hbm.at[pl.ds((row + c * _T) * _PIX, _PIX)],
                    plane_v.at[pl.ds(base + c * _PIX, _PIX)],
                    sem)
                for c in range(_C)]

        def fire(f, base, sem):
            for cp in in_copies(f, base, sem):
                cp.start()

        def drain(f, base, sem):
            for cp in in_copies(f, base, sem):
                cp.wait()

        def out_copy(f, hbase, sem):
            return pltpu.make_async_copy(
                hist_v.at[pl.ds(hbase, _NBINS)],
                hist_hbm.at[pl.ds(f * _NBINS, _NBINS)],
                sem)

        def do_frame(f, base, hbase, hsem, j):
            @pl.when(j > 0)
            def _():
                out_copy(f, hbase, hsem).wait()

            @plsc.parallel_loop(0, _NBINS // 16, unroll=4)
            def _(i):
                hist_v[pl.ds(hbase + i * 16, 16)] = zeros16

            @plsc.parallel_loop(0, _PIX, step=16, unroll=8)
            def _(i):
                off = base + i
                r = plane_v[pl.ds(off, 16)]
                gg = plane_v[pl.ds(off + _PIX, 16)]
                bb = plane_v[pl.ds(off + 2 * _PIX, 16)]
                bins = ((r & 0xE0) << 1) | ((gg & 0xE0) >> 2) | (bb >> 5)
                plsc.addupdate_scatter(
                    hist_v.at[pl.ds(hbase, _NBINS)], [bins], ones16)

            out_copy(f, hbase, hsem).start()

        fire(f0, 0, in_sem0)

        def pair_body(j, carry):
            f = f0 + 2 * j
            fire(f + 1, _PLANE3, in_sem1)
            drain(f, 0, in_sem0)
            do_frame(f, 0, 0, out_sem0, j)

            @pl.when(j < _FPW // 2 - 1)
            def _():
                fire(f + 2, 0, in_sem0)

            drain(f + 1, _PLANE3, in_sem1)
            do_frame(f + 1, _PLANE3, _NBINS, out_sem1, j)
            return carry

        lax.fori_loop(0, _FPW // 2, pair_body, 0)
        out_copy(f0 + _FPW - 2, 0, out_sem0).wait()
        out_copy(f0 + _FPW - 1, _NBINS, out_sem1).wait()

    return hist_kernel(frames_flat)


def _tc_post(hist3, w_pad, bias2):
    pad = (_LOOKUP - 1) // 2
    width = 512

    def body(x_ref, w_ref, b_ref, o_ref, sp_ref):
        x = x_ref[0]
        inv = 1.0 / jnp.sqrt(jnp.sum(x * x, axis=1, keepdims=True))
        xn = x * inv
        sims = lax.dot_general(xn, xn, (((1,), (1,)), ((), ())),
                               preferred_element_type=jnp.float32)
        sp_ref[...] = jnp.zeros((_T, width), jnp.float32)
        sp_ref[:, pad:pad + _T] = sims
        v = sp_ref[...]
        rows = lax.broadcasted_iota(jnp.int32, (_T, width), 0)
        for k in range(8):
            amt = 1 << k
            rolled = jnp.concatenate([v[:, amt:], v[:, :amt]], axis=1)
            v = jnp.where((rows & amt) != 0, rolled, v)
        band = v[:, :_OUT_DIM]
        out = lax.dot_general(band, w_ref[...], (((1,), (0,)), ((), ())),
                              preferred_element_type=jnp.float32)
        o_ref[0] = jnp.maximum(out + b_ref[...], 0.0)

    return pl.pallas_call(
        body,
        grid=(_B,),
        in_specs=[
            pl.BlockSpec((1, _T, _NBINS), lambda i: (i, 0, 0)),
            pl.BlockSpec((_OUT_DIM, _OUT_DIM), lambda i: (0, 0)),
            pl.BlockSpec((1, _OUT_DIM), lambda i: (0, 0)),
        ],
        out_specs=pl.BlockSpec((1, _T, _OUT_DIM), lambda i: (i, 0, 0)),
        out_shape=jax.ShapeDtypeStruct((_B, _T, _OUT_DIM), jnp.float32),
        scratch_shapes=[pltpu.VMEM((_T, width), jnp.float32)],
    )(hist3, w_pad, bias2)


def kernel(frames, Wfc, bfc):
    frames_flat = frames.reshape(_B * _C * _T * _PIX)
    hist = _sc_histograms(frames_flat)
    hist3 = hist.reshape(_B, _T, _NBINS)
    w_pad = jnp.pad(Wfc.T, ((0, _OUT_DIM - _LOOKUP), (0, 0)))
    bias2 = bfc.reshape(1, _OUT_DIM)
    return _tc_post(hist3, w_pad, bias2)
